# Initial kernel scaffold; baseline (speedup 1.0000x reference)
import jax
import jax.numpy as jnp
from jax.experimental import pallas as pl


def kernel(input_ids, position_ids, token_type_ids, word_emb, pos_emb, type_emb, gamma, beta):
    raise NotImplementedError("write your pallas kernel here")


# speedup vs baseline: 3.1072x; 3.1072x over previous
import functools

import jax
import jax.numpy as jnp
from jax import lax
from jax.experimental import pallas as pl
from jax.experimental.pallas import tpu as pltpu
from jax.experimental.pallas import tpu_sc as plsc

VOCAB = 1000000
HID = 64
MAXPOS = 512
NTYPES = 2
B = 4096
L = 200
N = B * L

NC = 2
NS = 16
NW = NC * NS
TOK_PER_W = N // NW
CHUNK = 128
CHUNKS = TOK_PER_W // CHUNK

_INV_HID = 1.0 / HID
_EPS = 1e-12


def _rsqrt(x):
    xi = lax.bitcast_convert_type(x, jnp.int32)
    yi = jnp.int32(0x5F3759DF) - lax.shift_right_arithmetic(xi, 1)
    y = lax.bitcast_convert_type(yi, jnp.float32)
    half = jnp.float32(0.5) * x
    for _ in range(3):
        y = y * (jnp.float32(1.5) - half * y * y)
    return y


def _sc_body(wids, pids, tids, word, pos, typ, gamma, beta, out,
             widx, pidx, tidx, cidx, wrows, ptrows,
             ptmp, ttmp, pttmp, gvec, bvec, pt_shared, sem_w, sem_pt):
    c = lax.axis_index("c")
    s = lax.axis_index("s")
    wid = c * NS + s
    base0 = wid * TOK_PER_W

    pltpu.sync_copy(pos.at[pl.ds(s * 32, 32)], ptmp)
    pltpu.sync_copy(typ, ttmp)
    pltpu.sync_copy(gamma, gvec)
    pltpu.sync_copy(beta, bvec)
    t0 = [ttmp[0, pl.ds(k * 16, 16)] for k in range(4)]
    t1 = [ttmp[1, pl.ds(k * 16, 16)] for k in range(4)]
    for r in range(32):
        for k in range(4):
            v = ptmp[r, pl.ds(k * 16, 16)]
            pttmp[2 * r, pl.ds(k * 16, 16)] = v + t0[k]
            pttmp[2 * r + 1, pl.ds(k * 16, 16)] = v + t1[k]
    pltpu.sync_copy(pttmp, pt_shared.at[pl.ds(s * 64, 64)])
    plsc.subcore_barrier()

    gv = [gvec[pl.ds(k * 16, 16)] for k in range(4)]
    bv = [bvec[pl.ds(k * 16, 16)] for k in range(4)]

    def chunk_body(i, carry):
        base = base0 + i * CHUNK
        pltpu.sync_copy(wids.at[pl.ds(base, CHUNK)], widx)
        pltpu.sync_copy(pids.at[pl.ds(base, CHUNK)], pidx)
        pltpu.sync_copy(tids.at[pl.ds(base, CHUNK)], tidx)
        for g in range(CHUNK // 16):
            p = pidx[pl.ds(g * 16, 16)]
            t = tidx[pl.ds(g * 16, 16)]
            cidx[pl.ds(g * 16, 16)] = p + p + t
        cw = pltpu.async_copy(word.at[widx], wrows, sem_w)
        cp = pltpu.async_copy(pt_shared.at[cidx], ptrows, sem_pt)
        cw.wait()
        cp.wait()

        def ln_body(j, carry2):
            for u in range(4):
                tok = j * 4 + u
                sv = [wrows[tok, pl.ds(k * 16, 16)]
                      + ptrows[tok, pl.ds(k * 16, 16)] for k in range(4)]
                tot = jnp.sum(sv[0] + sv[1] + sv[2] + sv[3])
                q = (sv[0] * sv[0] + sv[1] * sv[1]
                     + sv[2] * sv[2] + sv[3] * sv[3])
                ssq = jnp.sum(q)
                mu = tot * jnp.float32(_INV_HID)
                var = ssq * jnp.float32(_INV_HID) - mu * mu
                rstd = _rsqrt(var + jnp.float32(_EPS))
                for k in range(4):
                    wrows[tok, pl.ds(k * 16, 16)] = (
                        (sv[k] - mu) * rstd * gv[k] + bv[k])
            return carry2

        lax.fori_loop(0, CHUNK // 4, ln_body, 0, unroll=False)
        pltpu.sync_copy(wrows, out.at[pl.ds(base, CHUNK)])
        return carry

    lax.fori_loop(0, CHUNKS, chunk_body, 0, unroll=False)


@jax.jit
def _run(wids, pids, tids, word, pos, typ, gamma, beta):
    mesh = plsc.VectorSubcoreMesh(core_axis_name="c", subcore_axis_name="s")
    f = pl.kernel(
        _sc_body,
        out_type=jax.ShapeDtypeStruct((N, HID), jnp.float32),
        mesh=mesh,
        compiler_params=pltpu.CompilerParams(
            needs_layout_passes=False, use_tc_tiling_on_sc=False),
        scratch_types=[
            pltpu.VMEM((CHUNK,), jnp.int32),
            pltpu.VMEM((CHUNK,), jnp.int32),
            pltpu.VMEM((CHUNK,), jnp.int32),
            pltpu.VMEM((CHUNK,), jnp.int32),
            pltpu.VMEM((CHUNK, HID), jnp.float32),
            pltpu.VMEM((CHUNK, HID), jnp.float32),
            pltpu.VMEM((32, HID), jnp.float32),
            pltpu.VMEM((2, HID), jnp.float32),
            pltpu.VMEM((64, HID), jnp.float32),
            pltpu.VMEM((HID,), jnp.float32),
            pltpu.VMEM((HID,), jnp.float32),
            pltpu.VMEM_SHARED((MAXPOS * NTYPES, HID), jnp.float32),
            pltpu.SemaphoreType.DMA,
            pltpu.SemaphoreType.DMA,
        ],
    )
    return f(wids, pids, tids, word, pos, typ, gamma, beta)


def kernel(input_ids, position_ids, token_type_ids, word_emb, pos_emb,
           type_emb, gamma, beta):
    wids = input_ids.reshape(-1).astype(jnp.int32)
    pids = position_ids.reshape(-1).astype(jnp.int32)
    tids = token_type_ids.reshape(-1).astype(jnp.int32)
    out = _run(wids, pids, tids, word_emb, pos_emb, type_emb, gamma, beta)
    return out.reshape(B, L, HID)


# speedup vs baseline: 4.8241x; 1.5525x over previous
import jax
import jax.numpy as jnp
from jax import lax
from jax.experimental import pallas as pl
from jax.experimental.pallas import tpu as pltpu
from jax.experimental.pallas import tpu_sc as plsc

VOCAB = 1000000
HID = 64
MAXPOS = 512
NTYPES = 2
B = 4096
L = 200
N = B * L

NC = 2
NS = 16
NW = NC * NS
TOK_PER_W = N // NW
CHUNK = 128
CHUNKS = TOK_PER_W // CHUNK

_INV_HID = 1.0 / HID
_EPS = 1e-12


def _rsqrt(x):
    xi = lax.bitcast_convert_type(x, jnp.int32)
    yi = jnp.int32(0x5F3759DF) - lax.shift_right_arithmetic(xi, 1)
    y = lax.bitcast_convert_type(yi, jnp.float32)
    half = jnp.float32(0.5) * x
    for _ in range(3):
        y = y * (jnp.float32(1.5) - half * y * y)
    return y


def _sc_body(wids, pids, tids, word, pos, typ, gamma, beta, out,
             widx, pidx, tidx, cidxb, wrows, ptrows, orows,
             ptmp, ttmp, pttmp, gvec, bvec, pt_shared,
             sem_ids, sem_w, sem_pt, sem_out):
    c = lax.axis_index("c")
    s = lax.axis_index("s")
    wid = c * NS + s
    base0 = wid * TOK_PER_W

    pltpu.sync_copy(pos.at[pl.ds(s * 32, 32)], ptmp)
    pltpu.sync_copy(typ, ttmp)
    pltpu.sync_copy(gamma, gvec)
    pltpu.sync_copy(beta, bvec)
    t0 = [ttmp[0, pl.ds(k * 16, 16)] for k in range(4)]
    t1 = [ttmp[1, pl.ds(k * 16, 16)] for k in range(4)]
    for r in range(32):
        for k in range(4):
            v = ptmp[r, pl.ds(k * 16, 16)]
            pttmp[2 * r, pl.ds(k * 16, 16)] = v + t0[k]
            pttmp[2 * r + 1, pl.ds(k * 16, 16)] = v + t1[k]
    pltpu.sync_copy(pttmp, pt_shared.at[pl.ds(s * 64, 64)])
    plsc.subcore_barrier()

    gv = [gvec[pl.ds(k * 16, 16)] for k in range(4)]
    bv = [bvec[pl.ds(k * 16, 16)] for k in range(4)]

    def ids_start(cidx_, b):
        base = base0 + cidx_ * CHUNK
        pltpu.async_copy(wids.at[pl.ds(base, CHUNK)], widx.at[b], sem_ids[b])
        pltpu.async_copy(pids.at[pl.ds(base, CHUNK)], pidx.at[b], sem_ids[b])
        pltpu.async_copy(tids.at[pl.ds(base, CHUNK)], tidx.at[b], sem_ids[b])

    def ids_wait(b):
        pltpu.make_async_copy(
            wids.at[pl.ds(0, CHUNK)], widx.at[b], sem_ids[b]).wait()
        pltpu.make_async_copy(
            pids.at[pl.ds(0, CHUNK)], pidx.at[b], sem_ids[b]).wait()
        pltpu.make_async_copy(
            tids.at[pl.ds(0, CHUNK)], tidx.at[b], sem_ids[b]).wait()

    def cidx_compute(b):
        for g in range(CHUNK // 16):
            p = pidx[b, pl.ds(g * 16, 16)]
            t = tidx[b, pl.ds(g * 16, 16)]
            cidxb[b, pl.ds(g * 16, 16)] = p + p + t

    def gathers_start(b):
        pltpu.async_copy(word.at[widx.at[b]], wrows.at[b], sem_w[b])
        pltpu.async_copy(pt_shared.at[cidxb.at[b]], ptrows.at[b], sem_pt[b])

    def gathers_wait(b):
        pltpu.make_async_copy(
            word.at[widx.at[b]], wrows.at[b], sem_w[b]).wait()
        pltpu.make_async_copy(
            pt_shared.at[cidxb.at[b]], ptrows.at[b], sem_pt[b]).wait()

    def ln(b):
        @plsc.parallel_loop(0, CHUNK // 4)
        def _(j):
            for u in range(4):
                tok = j * 4 + u
                sv = [wrows[b, tok, pl.ds(k * 16, 16)]
                      + ptrows[b, tok, pl.ds(k * 16, 16)] for k in range(4)]
                tot = jnp.sum(sv[0] + sv[1] + sv[2] + sv[3])
                q = (sv[0] * sv[0] + sv[1] * sv[1]
                     + sv[2] * sv[2] + sv[3] * sv[3])
                ssq = jnp.sum(q)
                mu = tot * jnp.float32(_INV_HID)
                var = ssq * jnp.float32(_INV_HID) - mu * mu
                rstd = _rsqrt(var + jnp.float32(_EPS))
                for k in range(4):
                    orows[b, tok, pl.ds(k * 16, 16)] = (
                        (sv[k] - mu) * rstd * gv[k] + bv[k])

    def out_start(cidx_, b):
        base = base0 + cidx_ * CHUNK
        pltpu.async_copy(orows.at[b], out.at[pl.ds(base, CHUNK)], sem_out[b])

    def out_wait(b):
        pltpu.make_async_copy(
            orows.at[b], out.at[pl.ds(0, CHUNK)], sem_out[b]).wait()

    pltpu.sync_copy(wids.at[pl.ds(base0, CHUNK)], widx.at[0])
    pltpu.sync_copy(pids.at[pl.ds(base0, CHUNK)], pidx.at[0])
    pltpu.sync_copy(tids.at[pl.ds(base0, CHUNK)], tidx.at[0])
    cidx_compute(0)
    gathers_start(0)
    ids_start(1, 1)

    gathers_wait(0)
    ids_start(2, 0)
    ids_wait(1)
    cidx_compute(1)
    gathers_start(1)
    ln(0)
    out_start(0, 0)

    gathers_wait(1)
    ids_start(3, 1)
    ids_wait(0)
    cidx_compute(0)
    gathers_start(0)
    ln(1)
    out_start(1, 1)

    def steady(k, carry):
        cc = 2 + 2 * k
        for b in range(2):
            gathers_wait(b)
            ids_start(cc + b + 2, b)
            ids_wait(1 - b)
            cidx_compute(1 - b)
            gathers_start(1 - b)
            out_wait(b)
            ln(b)
            out_start(cc + b, b)
        return carry

    lax.fori_loop(0, (CHUNKS - 4) // 2, steady, 0, unroll=False)

    gathers_wait(0)
    ids_wait(1)
    cidx_compute(1)
    gathers_start(1)
    out_wait(0)
    ln(0)
    out_start(CHUNKS - 2, 0)

    gathers_wait(1)
    out_wait(1)
    ln(1)
    out_start(CHUNKS - 1, 1)

    out_wait(0)
    out_wait(1)


@jax.jit
def _run(wids, pids, tids, word, pos, typ, gamma, beta):
    mesh = plsc.VectorSubcoreMesh(core_axis_name="c", subcore_axis_name="s")
    f = pl.kernel(
        _sc_body,
        out_type=jax.ShapeDtypeStruct((N, HID), jnp.float32),
        mesh=mesh,
        compiler_params=pltpu.CompilerParams(
            needs_layout_passes=False, use_tc_tiling_on_sc=False),
        scratch_types=[
            pltpu.VMEM((2, CHUNK), jnp.int32),
            pltpu.VMEM((2, CHUNK), jnp.int32),
            pltpu.VMEM((2, CHUNK), jnp.int32),
            pltpu.VMEM((2, CHUNK), jnp.int32),
            pltpu.VMEM((2, CHUNK, HID), jnp.float32),
            pltpu.VMEM((2, CHUNK, HID), jnp.float32),
            pltpu.VMEM((2, CHUNK, HID), jnp.float32),
            pltpu.VMEM((32, HID), jnp.float32),
            pltpu.VMEM((2, HID), jnp.float32),
            pltpu.VMEM((64, HID), jnp.float32),
            pltpu.VMEM((HID,), jnp.float32),
            pltpu.VMEM((HID,), jnp.float32),
            pltpu.VMEM_SHARED((MAXPOS * NTYPES, HID), jnp.float32),
            [pltpu.SemaphoreType.DMA, pltpu.SemaphoreType.DMA],
            [pltpu.SemaphoreType.DMA, pltpu.SemaphoreType.DMA],
            [pltpu.SemaphoreType.DMA, pltpu.SemaphoreType.DMA],
            [pltpu.SemaphoreType.DMA, pltpu.SemaphoreType.DMA],
        ],
    )
    return f(wids, pids, tids, word, pos, typ, gamma, beta)


def kernel(input_ids, position_ids, token_type_ids, word_emb, pos_emb,
           type_emb, gamma, beta):
    wids = input_ids.reshape(-1).astype(jnp.int32)
    pids = position_ids.reshape(-1).astype(jnp.int32)
    tids = token_type_ids.reshape(-1).astype(jnp.int32)
    out = _run(wids, pids, tids, word_emb, pos_emb, type_emb, gamma, beta)
    return out.reshape(B, L, HID)


# speedup vs baseline: 4.8646x; 1.0084x over previous
import jax
import jax.numpy as jnp
from jax import lax
from jax.experimental import pallas as pl
from jax.experimental.pallas import tpu as pltpu
from jax.experimental.pallas import tpu_sc as plsc

VOCAB = 1000000
HID = 64
MAXPOS = 512
NTYPES = 2
B = 4096
L = 200

NC = 2
NS = 16
NW = NC * NS
RPW = B // NW

_INV_HID = 1.0 / HID
_EPS = 1e-12


def _rsqrt(x):
    xi = lax.bitcast_convert_type(x, jnp.int32)
    yi = jnp.int32(0x5F3759DF) - lax.shift_right_arithmetic(xi, 1)
    y = lax.bitcast_convert_type(yi, jnp.float32)
    half = jnp.float32(0.5) * x
    for _ in range(3):
        y = y * (jnp.float32(1.5) - half * y * y)
    return y


def _sc_body(wids, pids, tids, word, pos, typ, gamma, beta, out,
             widx, pidx, tidx, cidxb, wrows, ptrows, orows,
             ptmp, ttmp, pttmp, gvec, bvec, pt_shared,
             sem_ids, sem_w, sem_pt, sem_out):
    c = lax.axis_index("c")
    s = lax.axis_index("s")
    wid = c * NS + s
    row0 = wid * RPW

    pltpu.sync_copy(pos.at[pl.ds(s * 32, 32)], ptmp)
    pltpu.sync_copy(typ, ttmp)
    pltpu.sync_copy(gamma, gvec)
    pltpu.sync_copy(beta, bvec)
    t0 = [ttmp[0, pl.ds(k * 16, 16)] for k in range(4)]
    t1 = [ttmp[1, pl.ds(k * 16, 16)] for k in range(4)]
    for r in range(32):
        for k in range(4):
            v = ptmp[r, pl.ds(k * 16, 16)]
            pttmp[2 * r, pl.ds(k * 16, 16)] = v + t0[k]
            pttmp[2 * r + 1, pl.ds(k * 16, 16)] = v + t1[k]
    pltpu.sync_copy(pttmp, pt_shared.at[pl.ds(s * 64, 64)])
    plsc.subcore_barrier()

    gv = [gvec[pl.ds(k * 16, 16)] for k in range(4)]
    bv = [bvec[pl.ds(k * 16, 16)] for k in range(4)]

    def ids_start(r, b):
        pltpu.async_copy(wids.at[r], widx.at[b], sem_ids[b])
        pltpu.async_copy(pids.at[r], pidx.at[b], sem_ids[b])
        pltpu.async_copy(tids.at[r], tidx.at[b], sem_ids[b])

    def ids_wait(b):
        pltpu.make_async_copy(wids.at[0], widx.at[b], sem_ids[b]).wait()
        pltpu.make_async_copy(pids.at[0], pidx.at[b], sem_ids[b]).wait()
        pltpu.make_async_copy(tids.at[0], tidx.at[b], sem_ids[b]).wait()

    _GOFF = [g * 16 for g in range(12)] + [L - 16]

    def cidx_compute(b):
        for off in _GOFF:
            p = pidx[b, pl.ds(off, 16)]
            t = tidx[b, pl.ds(off, 16)]
            cidxb[b, pl.ds(off, 16)] = p + p + t

    _SPLITS = ((0, 128), (128, L - 128))

    def gathers_start(b):
        for off, n in _SPLITS:
            pltpu.async_copy(word.at[widx.at[b, pl.ds(off, n)]],
                             wrows.at[b, pl.ds(off, n)], sem_w[b])
            pltpu.async_copy(pt_shared.at[cidxb.at[b, pl.ds(off, n)]],
                             ptrows.at[b, pl.ds(off, n)], sem_pt[b])

    def gathers_wait(b):
        for off, n in _SPLITS:
            pltpu.make_async_copy(word.at[widx.at[b, pl.ds(off, n)]],
                                  wrows.at[b, pl.ds(off, n)], sem_w[b]).wait()
            pltpu.make_async_copy(pt_shared.at[cidxb.at[b, pl.ds(off, n)]],
                                  ptrows.at[b, pl.ds(off, n)],
                                  sem_pt[b]).wait()

    def ln(b):
        @plsc.parallel_loop(0, L // 4)
        def _(j):
            for u in range(4):
                tok = j * 4 + u
                sv = [wrows[b, tok, pl.ds(k * 16, 16)]
                      + ptrows[b, tok, pl.ds(k * 16, 16)] for k in range(4)]
                tot = jnp.sum(sv[0] + sv[1] + sv[2] + sv[3])
                q = (sv[0] * sv[0] + sv[1] * sv[1]
                     + sv[2] * sv[2] + sv[3] * sv[3])
                ssq = jnp.sum(q)
                mu = tot * jnp.float32(_INV_HID)
                var = ssq * jnp.float32(_INV_HID) - mu * mu
                rstd = _rsqrt(var + jnp.float32(_EPS))
                for k in range(4):
                    orows[b, tok, pl.ds(k * 16, 16)] = (
                        (sv[k] - mu) * rstd * gv[k] + bv[k])

    def out_start(r, b):
        pltpu.async_copy(orows.at[b], out.at[r], sem_out[b])

    def out_wait(b):
        pltpu.make_async_copy(orows.at[b], out.at[0], sem_out[b]).wait()

    pltpu.sync_copy(wids.at[row0], widx.at[0])
    pltpu.sync_copy(pids.at[row0], pidx.at[0])
    pltpu.sync_copy(tids.at[row0], tidx.at[0])
    cidx_compute(0)
    gathers_start(0)
    ids_start(row0 + 1, 1)

    gathers_wait(0)
    ids_start(row0 + 2, 0)
    ids_wait(1)
    cidx_compute(1)
    gathers_start(1)
    ln(0)
    out_start(row0, 0)

    gathers_wait(1)
    ids_start(row0 + 3, 1)
    ids_wait(0)
    cidx_compute(0)
    gathers_start(0)
    ln(1)
    out_start(row0 + 1, 1)

    def steady(k, carry):
        rr = row0 + 2 + 2 * k
        for b in range(2):
            gathers_wait(b)
            ids_start(rr + b + 2, b)
            ids_wait(1 - b)
            cidx_compute(1 - b)
            gathers_start(1 - b)
            out_wait(b)
            ln(b)
            out_start(rr + b, b)
        return carry

    lax.fori_loop(0, (RPW - 4) // 2, steady, 0, unroll=False)

    gathers_wait(0)
    ids_wait(1)
    cidx_compute(1)
    gathers_start(1)
    out_wait(0)
    ln(0)
    out_start(row0 + RPW - 2, 0)

    gathers_wait(1)
    out_wait(1)
    ln(1)
    out_start(row0 + RPW - 1, 1)

    out_wait(0)
    out_wait(1)


@jax.jit
def _run(wids, pids, tids, word, pos, typ, gamma, beta):
    mesh = plsc.VectorSubcoreMesh(core_axis_name="c", subcore_axis_name="s")
    f = pl.kernel(
        _sc_body,
        out_type=jax.ShapeDtypeStruct((B, L, HID), jnp.float32),
        mesh=mesh,
        compiler_params=pltpu.CompilerParams(
            needs_layout_passes=False, use_tc_tiling_on_sc=False),
        scratch_types=[
            pltpu.VMEM((2, L), jnp.int32),
            pltpu.VMEM((2, L), jnp.int32),
            pltpu.VMEM((2, L), jnp.int32),
            pltpu.VMEM((2, L), jnp.int32),
            pltpu.VMEM((2, L, HID), jnp.float32),
            pltpu.VMEM((2, L, HID), jnp.float32),
            pltpu.VMEM((2, L, HID), jnp.float32),
            pltpu.VMEM((32, HID), jnp.float32),
            pltpu.VMEM((2, HID), jnp.float32),
            pltpu.VMEM((64, HID), jnp.float32),
            pltpu.VMEM((HID,), jnp.float32),
            pltpu.VMEM((HID,), jnp.float32),
            pltpu.VMEM_SHARED((MAXPOS * NTYPES, HID), jnp.float32),
            [pltpu.SemaphoreType.DMA, pltpu.SemaphoreType.DMA],
            [pltpu.SemaphoreType.DMA, pltpu.SemaphoreType.DMA],
            [pltpu.SemaphoreType.DMA, pltpu.SemaphoreType.DMA],
            [pltpu.SemaphoreType.DMA, pltpu.SemaphoreType.DMA],
        ],
    )
    return f(wids, pids, tids, word, pos, typ, gamma, beta)


def kernel(input_ids, position_ids, token_type_ids, word_emb, pos_emb,
           type_emb, gamma, beta):
    return _run(input_ids.astype(jnp.int32), position_ids.astype(jnp.int32),
                token_type_ids.astype(jnp.int32), word_emb, pos_emb,
                type_emb, gamma, beta)


# speedup vs baseline: 5.9375x; 1.2205x over previous
import jax
import jax.numpy as jnp
from jax import lax
from jax.experimental import pallas as pl
from jax.experimental.pallas import tpu as pltpu
from jax.experimental.pallas import tpu_sc as plsc

VOCAB = 1000000
HID = 64
MAXPOS = 512
NTYPES = 2
B = 4096
L = 200

NC = 2
NS = 16
NW = NC * NS
RPW = B // NW

_INV_HID = 1.0 / HID
_EPS = 1e-12


def _rsqrt(x):
    xi = lax.bitcast_convert_type(x, jnp.int32)
    yi = jnp.int32(0x5F3759DF) - lax.shift_right_arithmetic(xi, 1)
    y = lax.bitcast_convert_type(yi, jnp.float32)
    half = jnp.float32(0.5) * x
    for _ in range(3):
        y = y * (jnp.float32(1.5) - half * y * y)
    return y


def _sc_body(wids, pids, tids, word, pos, typ, gamma, beta, out,
             widx, pidx, tidx, cidxb, wrows, ptrows, orows,
             ptmp, ttmp, pttmp, gvec, bvec, pt_shared,
             sem_ids, sem_w, sem_pt, sem_out):
    c = lax.axis_index("c")
    s = lax.axis_index("s")
    wid = c * NS + s
    row0 = wid * RPW

    pltpu.sync_copy(pos.at[pl.ds(s * 32, 32)], ptmp)
    pltpu.sync_copy(typ, ttmp)
    pltpu.sync_copy(gamma, gvec)
    pltpu.sync_copy(beta, bvec)
    t0 = [ttmp[0, pl.ds(k * 16, 16)] for k in range(4)]
    t1 = [ttmp[1, pl.ds(k * 16, 16)] for k in range(4)]
    for r in range(32):
        for k in range(4):
            v = ptmp[r, pl.ds(k * 16, 16)]
            pttmp[2 * r, pl.ds(k * 16, 16)] = v + t0[k]
            pttmp[2 * r + 1, pl.ds(k * 16, 16)] = v + t1[k]
    pltpu.sync_copy(pttmp, pt_shared.at[pl.ds(s * 64, 64)])
    plsc.subcore_barrier()

    gv = [gvec[pl.ds(k * 16, 16)] for k in range(4)]
    bv = [bvec[pl.ds(k * 16, 16)] for k in range(4)]

    def ids_start(r, b):
        pltpu.async_copy(wids.at[r], widx.at[b], sem_ids[b])
        pltpu.async_copy(pids.at[r], pidx.at[b], sem_ids[b])
        pltpu.async_copy(tids.at[r], tidx.at[b], sem_ids[b])

    def ids_wait(b):
        pltpu.make_async_copy(wids.at[0], widx.at[b], sem_ids[b]).wait()
        pltpu.make_async_copy(pids.at[0], pidx.at[b], sem_ids[b]).wait()
        pltpu.make_async_copy(tids.at[0], tidx.at[b], sem_ids[b]).wait()

    _GOFF = [g * 16 for g in range(12)] + [L - 16]

    def cidx_compute(b):
        for off in _GOFF:
            p = pidx[b, pl.ds(off, 16)]
            t = tidx[b, pl.ds(off, 16)]
            cidxb[b, pl.ds(off, 16)] = p + p + t

    _SPLITS = ((0, 128), (128, L - 128))

    def gathers_start(b):
        for off, n in _SPLITS:
            pltpu.async_copy(word.at[widx.at[b, pl.ds(off, n)]],
                             wrows.at[b, pl.ds(off, n)], sem_w[b])
            pltpu.async_copy(pt_shared.at[cidxb.at[b, pl.ds(off, n)]],
                             ptrows.at[b, pl.ds(off, n)], sem_pt[b])

    def gathers_wait(b):
        for off, n in _SPLITS:
            pltpu.make_async_copy(word.at[widx.at[b, pl.ds(off, n)]],
                                  wrows.at[b, pl.ds(off, n)], sem_w[b]).wait()
            pltpu.make_async_copy(pt_shared.at[cidxb.at[b, pl.ds(off, n)]],
                                  ptrows.at[b, pl.ds(off, n)],
                                  sem_pt[b]).wait()

    def ln(b):
        @plsc.parallel_loop(0, L // 4)
        def _(j):
            for u in range(4):
                tok = j * 4 + u
                sv = [wrows[b, tok, pl.ds(k * 16, 16)]
                      + ptrows[b, tok, pl.ds(k * 16, 16)] for k in range(4)]
                tot = jnp.sum(sv[0] + sv[1] + sv[2] + sv[3])
                q = (sv[0] * sv[0] + sv[1] * sv[1]
                     + sv[2] * sv[2] + sv[3] * sv[3])
                ssq = jnp.sum(q)
                mu = tot * jnp.float32(_INV_HID)
                var = ssq * jnp.float32(_INV_HID) - mu * mu
                rstd = _rsqrt(var + jnp.float32(_EPS))
                for k in range(4):
                    orows[b, tok, pl.ds(k * 16, 16)] = (
                        (sv[k] - mu) * rstd * gv[k] + bv[k])

    def out_start(r, b):
        pltpu.async_copy(orows.at[b], out.at[r], sem_out[b])

    def out_wait(b):
        pltpu.make_async_copy(orows.at[b], out.at[0], sem_out[b]).wait()

    def zero_pad(tok, carry):
        z = jnp.zeros((16,), jnp.float32)
        for bb in range(2):
            for k in range(4):
                orows[bb, tok, pl.ds(HID + k * 16, 16)] = z
        return carry

    lax.fori_loop(0, L, zero_pad, 0, unroll=False)

    pltpu.sync_copy(wids.at[row0], widx.at[0])
    pltpu.sync_copy(pids.at[row0], pidx.at[0])
    pltpu.sync_copy(tids.at[row0], tidx.at[0])
    cidx_compute(0)
    gathers_start(0)
    ids_start(row0 + 1, 1)

    gathers_wait(0)
    ids_start(row0 + 2, 0)
    ids_wait(1)
    cidx_compute(1)
    gathers_start(1)
    ln(0)
    out_start(row0, 0)

    gathers_wait(1)
    ids_start(row0 + 3, 1)
    ids_wait(0)
    cidx_compute(0)
    gathers_start(0)
    ln(1)
    out_start(row0 + 1, 1)

    def steady(k, carry):
        rr = row0 + 2 + 2 * k
        for b in range(2):
            gathers_wait(b)
            ids_start(rr + b + 2, b)
            ids_wait(1 - b)
            cidx_compute(1 - b)
            gathers_start(1 - b)
            out_wait(b)
            ln(b)
            out_start(rr + b, b)
        return carry

    lax.fori_loop(0, (RPW - 4) // 2, steady, 0, unroll=False)

    gathers_wait(0)
    ids_wait(1)
    cidx_compute(1)
    gathers_start(1)
    out_wait(0)
    ln(0)
    out_start(row0 + RPW - 2, 0)

    gathers_wait(1)
    out_wait(1)
    ln(1)
    out_start(row0 + RPW - 1, 1)

    out_wait(0)
    out_wait(1)


@jax.jit
def _run(wids, pids, tids, word, pos, typ, gamma, beta):
    mesh = plsc.VectorSubcoreMesh(core_axis_name="c", subcore_axis_name="s")
    f = pl.kernel(
        _sc_body,
        out_type=jax.ShapeDtypeStruct((B, L, 2 * HID), jnp.float32),
        mesh=mesh,
        compiler_params=pltpu.CompilerParams(
            needs_layout_passes=False, use_tc_tiling_on_sc=False),
        scratch_types=[
            pltpu.VMEM((2, L), jnp.int32),
            pltpu.VMEM((2, L), jnp.int32),
            pltpu.VMEM((2, L), jnp.int32),
            pltpu.VMEM((2, L), jnp.int32),
            pltpu.VMEM((2, L, HID), jnp.float32),
            pltpu.VMEM((2, L, HID), jnp.float32),
            pltpu.VMEM((2, L, 2 * HID), jnp.float32),
            pltpu.VMEM((32, HID), jnp.float32),
            pltpu.VMEM((2, HID), jnp.float32),
            pltpu.VMEM((64, HID), jnp.float32),
            pltpu.VMEM((HID,), jnp.float32),
            pltpu.VMEM((HID,), jnp.float32),
            pltpu.VMEM_SHARED((MAXPOS * NTYPES, HID), jnp.float32),
            [pltpu.SemaphoreType.DMA, pltpu.SemaphoreType.DMA],
            [pltpu.SemaphoreType.DMA, pltpu.SemaphoreType.DMA],
            [pltpu.SemaphoreType.DMA, pltpu.SemaphoreType.DMA],
            [pltpu.SemaphoreType.DMA, pltpu.SemaphoreType.DMA],
        ],
    )
    return f(wids, pids, tids, word, pos, typ, gamma, beta)


def kernel(input_ids, position_ids, token_type_ids, word_emb, pos_emb,
           type_emb, gamma, beta):
    word_lin = word_emb.reshape(-1).reshape(VOCAB, HID)
    out128 = _run(input_ids.astype(jnp.int32), position_ids.astype(jnp.int32),
                  token_type_ids.astype(jnp.int32), word_lin, pos_emb,
                  type_emb, gamma, beta)
    return out128[:, :, :HID]
